# initial kernel scaffold (unmeasured)
import jax
import jax.numpy as jnp
from jax import lax
from jax.experimental import pallas as pl
from jax.experimental.pallas import tpu as pltpu

M = 2048
D = 2048


def kernel(partial, gamma):
    x = partial[0]
    my_y = lax.axis_index("y")
    local = lax.dynamic_slice_in_dim(x, my_y * M, M, axis=0)
    to_send = lax.dynamic_slice_in_dim(x, (1 - my_y) * M, M, axis=0).astype(
        jnp.bfloat16
    )

    def body(local_ref, send_ref, gamma_ref, out_ref, recv_ref, send_sem, recv_sem):
        my_x = lax.axis_index("x")
        my_y = lax.axis_index("y")
        my_z = lax.axis_index("z")
        peer = (my_x, 1 - my_y, my_z)

        barrier_sem = pltpu.get_barrier_semaphore()
        pl.semaphore_signal(
            barrier_sem, inc=1, device_id=peer, device_id_type=pl.DeviceIdType.MESH
        )
        pl.semaphore_wait(barrier_sem, 1)

        rdma = pltpu.make_async_remote_copy(
            src_ref=send_ref,
            dst_ref=recv_ref,
            send_sem=send_sem,
            recv_sem=recv_sem,
            device_id=peer,
            device_id_type=pl.DeviceIdType.MESH,
        )
        rdma.start()
        rdma.wait()

        y = local_ref[...] + recv_ref[...].astype(jnp.float32)
        rms = jnp.sqrt(jnp.mean(y * y, axis=-1, keepdims=True) + 1e-6)
        out_ref[...] = y / rms * gamma_ref[...][None, :]

    return pl.pallas_call(
        body,
        out_shape=jax.ShapeDtypeStruct((M, D), jnp.float32),
        in_specs=[
            pl.BlockSpec(memory_space=pltpu.VMEM),
            pl.BlockSpec(memory_space=pltpu.VMEM),
            pl.BlockSpec(memory_space=pltpu.VMEM),
        ],
        out_specs=pl.BlockSpec(memory_space=pltpu.VMEM),
        scratch_shapes=[
            pltpu.VMEM((M, D), jnp.bfloat16),
            pltpu.SemaphoreType.DMA,
            pltpu.SemaphoreType.DMA,
        ],
        compiler_params=pltpu.CompilerParams(collective_id=0),
    )(local, to_send, gamma)


# baseline (device time: 128316 ns/iter reference)
import jax
import jax.numpy as jnp
from jax import lax
from jax.experimental import pallas as pl
from jax.experimental.pallas import tpu as pltpu

M = 2048
D = 2048
BLK = 256


def _comm(to_send):

    def body(send_ref, recv_ref, send_sem, recv_sem):
        my_x = lax.axis_index("x")
        my_y = lax.axis_index("y")
        my_z = lax.axis_index("z")
        peer = (my_x, 1 - my_y, my_z)

        barrier_sem = pltpu.get_barrier_semaphore()
        pl.semaphore_signal(
            barrier_sem, inc=1, device_id=peer, device_id_type=pl.DeviceIdType.MESH
        )
        pl.semaphore_wait(barrier_sem, 1)

        rdma = pltpu.make_async_remote_copy(
            src_ref=send_ref,
            dst_ref=recv_ref,
            send_sem=send_sem,
            recv_sem=recv_sem,
            device_id=peer,
            device_id_type=pl.DeviceIdType.MESH,
        )
        rdma.start()
        rdma.wait()

    return pl.pallas_call(
        body,
        out_shape=jax.ShapeDtypeStruct((M, D), jnp.bfloat16),
        in_specs=[pl.BlockSpec(memory_space=pl.ANY)],
        out_specs=pl.BlockSpec(memory_space=pl.ANY),
        scratch_shapes=[
            pltpu.SemaphoreType.DMA,
            pltpu.SemaphoreType.DMA,
        ],
        compiler_params=pltpu.CompilerParams(collective_id=0),
    )(to_send)


def _fused_norm(local, recv, gamma):

    def body(local_ref, recv_ref, gamma_ref, out_ref):
        y = local_ref[...] + recv_ref[...].astype(jnp.float32)
        rms = jnp.sqrt(jnp.mean(y * y, axis=-1, keepdims=True) + 1e-6)
        out_ref[...] = y / rms * gamma_ref[...][None, :]

    return pl.pallas_call(
        body,
        grid=(M // BLK,),
        out_shape=jax.ShapeDtypeStruct((M, D), jnp.float32),
        in_specs=[
            pl.BlockSpec((BLK, D), lambda i: (i, 0)),
            pl.BlockSpec((BLK, D), lambda i: (i, 0)),
            pl.BlockSpec((D,), lambda i: (0,)),
        ],
        out_specs=pl.BlockSpec((BLK, D), lambda i: (i, 0)),
    )(local, recv, gamma)


def kernel(partial, gamma):
    x = partial[0]
    my_y = lax.axis_index("y")
    local = lax.dynamic_slice_in_dim(x, my_y * M, M, axis=0)
    to_send = lax.dynamic_slice_in_dim(x, (1 - my_y) * M, M, axis=0).astype(
        jnp.bfloat16
    )
    recv = _comm(to_send)
    return _fused_norm(local, recv, gamma)


# device time: 64834 ns/iter; 1.9791x vs baseline; 1.9791x over previous
import jax
import jax.numpy as jnp
from jax import lax
from jax.experimental import pallas as pl
from jax.experimental.pallas import tpu as pltpu

M = 2048
D = 2048
H = M // 2
C = 128
NC = H // C


def kernel(partial, gamma):
    x = partial.reshape(2 * M, D)

    def body(
        x_hbm,
        gamma_ref,
        out_hbm,
        stage,
        send_buf,
        recv_y,
        recv_x,
        local_buf,
        out_buf,
        stage_sem,
        local_sem,
        out_sem,
        ysend_sem,
        yrecv_sem,
        xsend_sem,
        xrecv_sem,
    ):
        my_x = lax.axis_index("x")
        my_y = lax.axis_index("y")
        my_z = lax.axis_index("z")
        ypeer = (my_x, 1 - my_y, my_z)
        xpeer = (1 - my_x, my_y, my_z)

        blk0 = my_y * M
        send0 = (1 - my_y) * M + my_x * H
        mine0 = blk0 + my_x * H
        oth0 = blk0 + (1 - my_x) * H

        def stage_cp(k, slot):
            return pltpu.make_async_copy(
                x_hbm.at[pl.ds(send0 + k * C, C), :],
                stage.at[slot],
                stage_sem.at[slot],
            )

        def local_cp(j, slot):
            if j < NC:
                row = mine0 + j * C
            else:
                row = oth0 + (j - NC) * C
            return pltpu.make_async_copy(
                x_hbm.at[pl.ds(row, C), :],
                local_buf.at[slot],
                local_sem.at[slot],
            )

        stage_cp(0, 0).start()
        stage_cp(1, 1).start()
        local_cp(0, 0).start()
        local_cp(1, 1).start()

        barrier = pltpu.get_barrier_semaphore()
        for p in (ypeer, xpeer):
            pl.semaphore_signal(
                barrier, inc=1, device_id=p, device_id_type=pl.DeviceIdType.MESH
            )
        pl.semaphore_wait(barrier, 2)

        rdma_y = []
        for k in range(NC):
            stage_cp(k, k % 2).wait()
            send_buf[k] = stage[k % 2].astype(jnp.bfloat16)
            r = pltpu.make_async_remote_copy(
                src_ref=send_buf.at[k],
                dst_ref=recv_y.at[k],
                send_sem=ysend_sem.at[k],
                recv_sem=yrecv_sem.at[k],
                device_id=ypeer,
                device_id_type=pl.DeviceIdType.MESH,
            )
            r.start()
            rdma_y.append(r)
            if k + 2 < NC:
                stage_cp(k + 2, k % 2).start()

        out_cps = [None, None]

        def compute(j, recv_ref, k):
            slot = j % 2
            local_cp(j, slot).wait()
            yv = local_buf[slot] + recv_ref[k].astype(jnp.float32)
            rms = jnp.sqrt(jnp.mean(yv * yv, axis=-1, keepdims=True) + 1e-6)
            if out_cps[slot] is not None:
                out_cps[slot].wait()
            out_buf[slot] = yv / rms * gamma_ref[...][None, :]
            if j < NC:
                orow = my_x * H + j * C
            else:
                orow = (1 - my_x) * H + (j - NC) * C
            cp = pltpu.make_async_copy(
                out_buf.at[slot], out_hbm.at[pl.ds(orow, C), :], out_sem.at[slot]
            )
            cp.start()
            out_cps[slot] = cp
            if j + 2 < 2 * NC:
                local_cp(j + 2, slot).start()

        rdma_x = []
        for k in range(NC):
            rdma_y[k].wait_recv()
            r = pltpu.make_async_remote_copy(
                src_ref=recv_y.at[k],
                dst_ref=recv_x.at[k],
                send_sem=xsend_sem.at[k],
                recv_sem=xrecv_sem.at[k],
                device_id=xpeer,
                device_id_type=pl.DeviceIdType.MESH,
            )
            r.start()
            rdma_x.append(r)
            compute(k, recv_y, k)

        for k in range(NC):
            rdma_x[k].wait_recv()
            compute(NC + k, recv_x, k)

        for k in range(NC):
            rdma_y[k].wait_send()
            rdma_x[k].wait_send()
        out_cps[0].wait()
        out_cps[1].wait()

    return pl.pallas_call(
        body,
        out_shape=jax.ShapeDtypeStruct((M, D), jnp.float32),
        in_specs=[
            pl.BlockSpec(memory_space=pl.ANY),
            pl.BlockSpec(memory_space=pltpu.VMEM),
        ],
        out_specs=pl.BlockSpec(memory_space=pl.ANY),
        scratch_shapes=[
            pltpu.VMEM((2, C, D), jnp.float32),
            pltpu.VMEM((NC, C, D), jnp.bfloat16),
            pltpu.VMEM((NC, C, D), jnp.bfloat16),
            pltpu.VMEM((NC, C, D), jnp.bfloat16),
            pltpu.VMEM((2, C, D), jnp.float32),
            pltpu.VMEM((2, C, D), jnp.float32),
            pltpu.SemaphoreType.DMA((2,)),
            pltpu.SemaphoreType.DMA((2,)),
            pltpu.SemaphoreType.DMA((2,)),
            pltpu.SemaphoreType.DMA((NC,)),
            pltpu.SemaphoreType.DMA((NC,)),
            pltpu.SemaphoreType.DMA((NC,)),
            pltpu.SemaphoreType.DMA((NC,)),
        ],
        compiler_params=pltpu.CompilerParams(collective_id=0),
    )(x, gamma)


# device time: 55238 ns/iter; 2.3230x vs baseline; 1.1737x over previous
import jax
import jax.numpy as jnp
from jax import lax
from jax.experimental import pallas as pl
from jax.experimental.pallas import tpu as pltpu

M = 2048
D = 2048
Q = M // 4
C = 128
NQ = Q // C


def kernel(partial, gamma):
    x = partial.reshape(2 * M, D)

    def body(
        x_hbm,
        gamma_ref,
        out_hbm,
        stage,
        send_buf,
        recv_y,
        recv_x,
        recv_z,
        recv_xd,
        recv_zd,
        local_buf,
        out_buf,
        stage_sem,
        local_sem,
        out_sem,
        ysend_sem,
        yrecv_sem,
        xfsend_sem,
        xfrecv_sem,
        zfsend_sem,
        zfrecv_sem,
        xrsend_sem,
        xrrecv_sem,
        zrsend_sem,
        zrrecv_sem,
    ):
        my_x = lax.axis_index("x")
        my_y = lax.axis_index("y")
        my_z = lax.axis_index("z")
        zq = lax.rem(my_z, 2)
        zpz = my_z + 1 - 2 * zq
        ypeer = (my_x, 1 - my_y, my_z)
        xpeer = (1 - my_x, my_y, my_z)
        zpeer = (my_x, my_y, zpz)

        blk0 = my_y * M
        qf0 = (2 * my_x + zq) * Q
        qx0 = (2 * (1 - my_x) + zq) * Q
        qz0 = (2 * my_x + (1 - zq)) * Q
        qd0 = (2 * (1 - my_x) + (1 - zq)) * Q
        send0 = (1 - my_y) * M + qf0

        rows_seq = [qf0 + k * C for k in range(NQ)]
        recv_seq = [(recv_y, k) for k in range(NQ)]
        for k in range(NQ):
            rows_seq += [qx0 + k * C, qz0 + k * C]
            recv_seq += [(recv_x, k), (recv_z, k)]
        rows_seq += [qd0 + k * C for k in range(NQ)]
        recv_seq += [(recv_xd, 0), (recv_xd, 1), (recv_zd, 0), (recv_zd, 1)]
        NTOT = len(rows_seq)

        def stage_cp(k, slot):
            return pltpu.make_async_copy(
                x_hbm.at[pl.ds(send0 + k * C, C), :],
                stage.at[slot],
                stage_sem.at[slot],
            )

        def local_cp(j, slot):
            return pltpu.make_async_copy(
                x_hbm.at[pl.ds(blk0 + rows_seq[j], C), :],
                local_buf.at[slot],
                local_sem.at[slot],
            )

        stage_cp(0, 0).start()
        stage_cp(1, 1).start()
        local_cp(0, 0).start()
        local_cp(1, 1).start()

        barrier = pltpu.get_barrier_semaphore()
        for p in (ypeer, xpeer, zpeer):
            pl.semaphore_signal(
                barrier, inc=1, device_id=p, device_id_type=pl.DeviceIdType.MESH
            )
        pl.semaphore_wait(barrier, 3)

        rdma_y = []
        for k in range(NQ):
            stage_cp(k, k % 2).wait()
            send_buf[k] = stage[k % 2].astype(jnp.bfloat16)
            r = pltpu.make_async_remote_copy(
                src_ref=send_buf.at[k],
                dst_ref=recv_y.at[k],
                send_sem=ysend_sem.at[k],
                recv_sem=yrecv_sem.at[k],
                device_id=ypeer,
                device_id_type=pl.DeviceIdType.MESH,
            )
            r.start()
            rdma_y.append(r)
            if k + 2 < NQ:
                stage_cp(k + 2, k % 2).start()

        out_cps = [None, None]

        def compute(j):
            slot = j % 2
            local_cp(j, slot).wait()
            ref, k = recv_seq[j]
            yv = local_buf[slot] + ref[k].astype(jnp.float32)
            rms = jnp.sqrt(jnp.mean(yv * yv, axis=-1, keepdims=True) + 1e-6)
            if out_cps[slot] is not None:
                out_cps[slot].wait()
            out_buf[slot] = yv / rms * gamma_ref[...][None, :]
            cp = pltpu.make_async_copy(
                out_buf.at[slot],
                out_hbm.at[pl.ds(rows_seq[j], C), :],
                out_sem.at[slot],
            )
            cp.start()
            out_cps[slot] = cp
            if j + 2 < NTOT:
                local_cp(j + 2, slot).start()

        fx, fz = [], []
        j = 0
        for k in range(NQ):
            rdma_y[k].wait_recv()
            r = pltpu.make_async_remote_copy(
                src_ref=recv_y.at[k],
                dst_ref=recv_x.at[k],
                send_sem=xfsend_sem.at[k],
                recv_sem=xfrecv_sem.at[k],
                device_id=xpeer,
                device_id_type=pl.DeviceIdType.MESH,
            )
            r.start()
            fx.append(r)
            r = pltpu.make_async_remote_copy(
                src_ref=recv_y.at[k],
                dst_ref=recv_z.at[k],
                send_sem=zfsend_sem.at[k],
                recv_sem=zfrecv_sem.at[k],
                device_id=zpeer,
                device_id_type=pl.DeviceIdType.MESH,
            )
            r.start()
            fz.append(r)
            compute(j)
            j += 1

        xr, zr = [], []
        for k in range(NQ):
            fx[k].wait_recv()
            if k >= 2:
                r = pltpu.make_async_remote_copy(
                    src_ref=recv_x.at[k],
                    dst_ref=recv_zd.at[k - 2],
                    send_sem=zrsend_sem.at[k - 2],
                    recv_sem=zrrecv_sem.at[k - 2],
                    device_id=zpeer,
                    device_id_type=pl.DeviceIdType.MESH,
                )
                r.start()
                zr.append(r)
            compute(j)
            j += 1
            fz[k].wait_recv()
            if k < 2:
                r = pltpu.make_async_remote_copy(
                    src_ref=recv_z.at[k],
                    dst_ref=recv_xd.at[k],
                    send_sem=xrsend_sem.at[k],
                    recv_sem=xrrecv_sem.at[k],
                    device_id=xpeer,
                    device_id_type=pl.DeviceIdType.MESH,
                )
                r.start()
                xr.append(r)
            compute(j)
            j += 1

        for k in range(2):
            xr[k].wait_recv()
            compute(j)
            j += 1
        for k in range(2):
            zr[k].wait_recv()
            compute(j)
            j += 1

        for k in range(NQ):
            rdma_y[k].wait_send()
            fx[k].wait_send()
            fz[k].wait_send()
        for k in range(2):
            xr[k].wait_send()
            zr[k].wait_send()
        out_cps[0].wait()
        out_cps[1].wait()

    return pl.pallas_call(
        body,
        out_shape=jax.ShapeDtypeStruct((M, D), jnp.float32),
        in_specs=[
            pl.BlockSpec(memory_space=pl.ANY),
            pl.BlockSpec(memory_space=pltpu.VMEM),
        ],
        out_specs=pl.BlockSpec(memory_space=pl.ANY),
        scratch_shapes=[
            pltpu.VMEM((2, C, D), jnp.float32),
            pltpu.VMEM((NQ, C, D), jnp.bfloat16),
            pltpu.VMEM((NQ, C, D), jnp.bfloat16),
            pltpu.VMEM((NQ, C, D), jnp.bfloat16),
            pltpu.VMEM((NQ, C, D), jnp.bfloat16),
            pltpu.VMEM((2, C, D), jnp.bfloat16),
            pltpu.VMEM((2, C, D), jnp.bfloat16),
            pltpu.VMEM((2, C, D), jnp.float32),
            pltpu.VMEM((2, C, D), jnp.float32),
            pltpu.SemaphoreType.DMA((2,)),
            pltpu.SemaphoreType.DMA((2,)),
            pltpu.SemaphoreType.DMA((2,)),
            pltpu.SemaphoreType.DMA((NQ,)),
            pltpu.SemaphoreType.DMA((NQ,)),
            pltpu.SemaphoreType.DMA((NQ,)),
            pltpu.SemaphoreType.DMA((NQ,)),
            pltpu.SemaphoreType.DMA((NQ,)),
            pltpu.SemaphoreType.DMA((NQ,)),
            pltpu.SemaphoreType.DMA((2,)),
            pltpu.SemaphoreType.DMA((2,)),
            pltpu.SemaphoreType.DMA((2,)),
            pltpu.SemaphoreType.DMA((2,)),
        ],
        compiler_params=pltpu.CompilerParams(collective_id=0),
    )(x, gamma)


# device time: 53806 ns/iter; 2.3848x vs baseline; 1.0266x over previous
import jax
import jax.numpy as jnp
from jax import lax
from jax.experimental import pallas as pl
from jax.experimental.pallas import tpu as pltpu

M = 2048
D = 2048
Q = M // 4
C = 64
NQ = Q // C
NH = NQ // 2


def kernel(partial, gamma):
    x = partial.reshape(2 * M, D)

    def body(
        x_hbm,
        gamma_ref,
        out_hbm,
        stage,
        send_buf,
        recv_y,
        recv_x,
        recv_z,
        recv_xd,
        recv_zd,
        local_buf,
        out_buf,
        stage_sem,
        local_sem,
        out_sem,
        ysend_sem,
        yrecv_sem,
        xfsend_sem,
        xfrecv_sem,
        zfsend_sem,
        zfrecv_sem,
        xrsend_sem,
        xrrecv_sem,
        zrsend_sem,
        zrrecv_sem,
    ):
        my_x = lax.axis_index("x")
        my_y = lax.axis_index("y")
        my_z = lax.axis_index("z")
        zq = lax.rem(my_z, 2)
        zpz = my_z + 1 - 2 * zq
        ypeer = (my_x, 1 - my_y, my_z)
        xpeer = (1 - my_x, my_y, my_z)
        zpeer = (my_x, my_y, zpz)

        blk0 = my_y * M
        qf0 = (2 * my_x + zq) * Q
        qx0 = (2 * (1 - my_x) + zq) * Q
        qz0 = (2 * my_x + (1 - zq)) * Q
        qd0 = (2 * (1 - my_x) + (1 - zq)) * Q
        send0 = (1 - my_y) * M + qf0

        rows_seq = [qf0 + k * C for k in range(NQ)]
        recv_seq = [(recv_y, k) for k in range(NQ)]
        for k in range(NQ):
            rows_seq += [qx0 + k * C, qz0 + k * C]
            recv_seq += [(recv_x, k), (recv_z, k)]
        rows_seq += [qd0 + k * C for k in range(NQ)]
        recv_seq += [(recv_xd, k) for k in range(NH)]
        recv_seq += [(recv_zd, k) for k in range(NH)]
        NTOT = len(rows_seq)

        def stage_cp(k, slot):
            return pltpu.make_async_copy(
                x_hbm.at[pl.ds(send0 + k * C, C), :],
                stage.at[slot],
                stage_sem.at[slot],
            )

        def local_cp(j, slot):
            return pltpu.make_async_copy(
                x_hbm.at[pl.ds(blk0 + rows_seq[j], C), :],
                local_buf.at[slot],
                local_sem.at[slot],
            )

        stage_cp(0, 0).start()
        stage_cp(1, 1).start()
        local_cp(0, 0).start()
        local_cp(1, 1).start()

        barrier = pltpu.get_barrier_semaphore()
        for p in (ypeer, xpeer, zpeer):
            pl.semaphore_signal(
                barrier, inc=1, device_id=p, device_id_type=pl.DeviceIdType.MESH
            )
        pl.semaphore_wait(barrier, 3)

        rdma_y = []
        for k in range(NQ):
            stage_cp(k, k % 2).wait()
            send_buf[k] = stage[k % 2].astype(jnp.bfloat16)
            r = pltpu.make_async_remote_copy(
                src_ref=send_buf.at[k],
                dst_ref=recv_y.at[k],
                send_sem=ysend_sem.at[k],
                recv_sem=yrecv_sem.at[k],
                device_id=ypeer,
                device_id_type=pl.DeviceIdType.MESH,
            )
            r.start()
            rdma_y.append(r)
            if k + 2 < NQ:
                stage_cp(k + 2, k % 2).start()

        out_cps = [None, None]

        def compute(j):
            slot = j % 2
            local_cp(j, slot).wait()
            ref, k = recv_seq[j]
            yv = local_buf[slot] + ref[k].astype(jnp.float32)
            rms = jnp.sqrt(jnp.mean(yv * yv, axis=-1, keepdims=True) + 1e-6)
            if out_cps[slot] is not None:
                out_cps[slot].wait()
            out_buf[slot] = yv / rms * gamma_ref[...][None, :]
            cp = pltpu.make_async_copy(
                out_buf.at[slot],
                out_hbm.at[pl.ds(rows_seq[j], C), :],
                out_sem.at[slot],
            )
            cp.start()
            out_cps[slot] = cp
            if j + 2 < NTOT:
                local_cp(j + 2, slot).start()

        fx, fz = [], []
        j = 0
        for k in range(NQ):
            rdma_y[k].wait_recv()
            r = pltpu.make_async_remote_copy(
                src_ref=recv_y.at[k],
                dst_ref=recv_x.at[k],
                send_sem=xfsend_sem.at[k],
                recv_sem=xfrecv_sem.at[k],
                device_id=xpeer,
                device_id_type=pl.DeviceIdType.MESH,
            )
            r.start()
            fx.append(r)
            r = pltpu.make_async_remote_copy(
                src_ref=recv_y.at[k],
                dst_ref=recv_z.at[k],
                send_sem=zfsend_sem.at[k],
                recv_sem=zfrecv_sem.at[k],
                device_id=zpeer,
                device_id_type=pl.DeviceIdType.MESH,
            )
            r.start()
            fz.append(r)
            compute(j)
            j += 1

        xr, zr = [], []
        for k in range(NQ):
            fx[k].wait_recv()
            if k >= NH:
                r = pltpu.make_async_remote_copy(
                    src_ref=recv_x.at[k],
                    dst_ref=recv_zd.at[k - NH],
                    send_sem=zrsend_sem.at[k - NH],
                    recv_sem=zrrecv_sem.at[k - NH],
                    device_id=zpeer,
                    device_id_type=pl.DeviceIdType.MESH,
                )
                r.start()
                zr.append(r)
            compute(j)
            j += 1
            fz[k].wait_recv()
            if k < NH:
                r = pltpu.make_async_remote_copy(
                    src_ref=recv_z.at[k],
                    dst_ref=recv_xd.at[k],
                    send_sem=xrsend_sem.at[k],
                    recv_sem=xrrecv_sem.at[k],
                    device_id=xpeer,
                    device_id_type=pl.DeviceIdType.MESH,
                )
                r.start()
                xr.append(r)
            compute(j)
            j += 1

        for k in range(NH):
            xr[k].wait_recv()
            compute(j)
            j += 1
        for k in range(NH):
            zr[k].wait_recv()
            compute(j)
            j += 1

        for k in range(NQ):
            rdma_y[k].wait_send()
            fx[k].wait_send()
            fz[k].wait_send()
        for k in range(NH):
            xr[k].wait_send()
            zr[k].wait_send()
        out_cps[0].wait()
        out_cps[1].wait()

    return pl.pallas_call(
        body,
        out_shape=jax.ShapeDtypeStruct((M, D), jnp.float32),
        in_specs=[
            pl.BlockSpec(memory_space=pl.ANY),
            pl.BlockSpec(memory_space=pltpu.VMEM),
        ],
        out_specs=pl.BlockSpec(memory_space=pl.ANY),
        scratch_shapes=[
            pltpu.VMEM((2, C, D), jnp.float32),
            pltpu.VMEM((NQ, C, D), jnp.bfloat16),
            pltpu.VMEM((NQ, C, D), jnp.bfloat16),
            pltpu.VMEM((NQ, C, D), jnp.bfloat16),
            pltpu.VMEM((NQ, C, D), jnp.bfloat16),
            pltpu.VMEM((NH, C, D), jnp.bfloat16),
            pltpu.VMEM((NH, C, D), jnp.bfloat16),
            pltpu.VMEM((2, C, D), jnp.float32),
            pltpu.VMEM((2, C, D), jnp.float32),
            pltpu.SemaphoreType.DMA((2,)),
            pltpu.SemaphoreType.DMA((2,)),
            pltpu.SemaphoreType.DMA((2,)),
            pltpu.SemaphoreType.DMA((NQ,)),
            pltpu.SemaphoreType.DMA((NQ,)),
            pltpu.SemaphoreType.DMA((NQ,)),
            pltpu.SemaphoreType.DMA((NQ,)),
            pltpu.SemaphoreType.DMA((NQ,)),
            pltpu.SemaphoreType.DMA((NQ,)),
            pltpu.SemaphoreType.DMA((NH,)),
            pltpu.SemaphoreType.DMA((NH,)),
            pltpu.SemaphoreType.DMA((NH,)),
            pltpu.SemaphoreType.DMA((NH,)),
        ],
        compiler_params=pltpu.CompilerParams(collective_id=0),
    )(x, gamma)
